# baseline (speedup 1.0000x reference)
import functools

import jax
import jax.numpy as jnp
from jax.experimental import pallas as pl
from jax.experimental.pallas import tpu as pltpu
from jax.experimental.pallas import tpu_sc as plsc

EMB_DIM = 64
NUM_CODES = 1024
N_ROWS = 16 * 1024
N_CHUNKS = 2
CHUNK_ROWS = N_ROWS // N_CHUNKS
ROW_TILE = 2048
GATHER_WINDOW = 128
GATHER_WIDTH = 128


def _vq_tc_body(n_tiles, z_ref, cb_ref, zsq_ref, csq_ref, idx_ref, loss_ref):
    z = z_ref[...]
    cb = cb_ref[...]
    dots = jax.lax.dot_general(z, cb, (((1,), (1,)), ((), ())),
                               preferred_element_type=jnp.float32)
    d2 = zsq_ref[...] + csq_ref[...] - 2.0 * dots
    dist = jnp.sqrt(jnp.clip(d2, 0.0, None))
    dmin = jnp.min(dist, axis=1, keepdims=True)
    ids = jax.lax.broadcasted_iota(jnp.int32, dist.shape, 1)
    idx = jnp.min(jnp.where(dist == dmin, ids, NUM_CODES), axis=1,
                  keepdims=True)
    idx_ref[...] = idx
    part = jnp.sum(dmin * dmin, keepdims=True)

    @pl.when(pl.program_id(0) == 0)
    def _init():
        loss_ref[...] = jnp.zeros_like(part)

    loss_ref[...] += part


def _vq_distances_argmin(z_chunk, codebook, csq):
    n_rows = z_chunk.shape[0]
    n_tiles = n_rows // ROW_TILE
    zsq = jnp.sum(z_chunk ** 2, axis=1, keepdims=True)
    return pl.pallas_call(
        functools.partial(_vq_tc_body, n_tiles),
        grid=(n_tiles,),
        in_specs=[
            pl.BlockSpec((ROW_TILE, EMB_DIM), lambda i: (i, 0)),
            pl.BlockSpec((NUM_CODES, EMB_DIM), lambda i: (0, 0)),
            pl.BlockSpec((ROW_TILE, 1), lambda i: (i, 0)),
            pl.BlockSpec((1, NUM_CODES), lambda i: (0, 0)),
        ],
        out_specs=[
            pl.BlockSpec((ROW_TILE, 1), lambda i: (i, 0)),
            pl.BlockSpec((1, 1), lambda i: (0, 0)),
        ],
        out_shape=[
            jax.ShapeDtypeStruct((n_rows, 1), jnp.int32),
            jax.ShapeDtypeStruct((1, 1), jnp.float32),
        ],
    )(z_chunk, codebook, zsq, csq)


def _sc_gather(codebook_padded, indices_2d):
    n_idx = indices_2d.shape[1]
    mesh = plsc.VectorSubcoreMesh(core_axis_name="core",
                                  subcore_axis_name="subcore")

    @pl.kernel(out_type=jax.ShapeDtypeStruct((n_idx, GATHER_WIDTH),
                                             jnp.float32),
               mesh=mesh)
    def gather_kernel(cb_hbm, i_hbm, o_hbm):
        def body(i_vmem, o_vmem):
            pltpu.sync_copy(cb_hbm.at[i_vmem.at[0]], o_vmem)

        pltpu.emit_pipeline(
            body,
            grid=(n_idx // GATHER_WINDOW,),
            in_specs=[pl.BlockSpec((1, GATHER_WINDOW),
                                   index_map=lambda i: (0, i))],
            out_specs=[pl.BlockSpec((GATHER_WINDOW, GATHER_WIDTH),
                                    index_map=lambda i: (i, 0))],
            core_axis_name=("core", "subcore"),
            dimension_semantics=(pltpu.PARALLEL,),
        )(i_hbm, o_hbm)

    return gather_kernel(codebook_padded, indices_2d)


def kernel(z, codebook):
    z_flat = z.reshape(-1, EMB_DIM)
    csq = jnp.sum(codebook ** 2, axis=1)[None, :]
    cb_padded = jnp.pad(codebook, ((0, 0), (0, GATHER_WIDTH - EMB_DIM)))

    idx_chunks, loss_parts, zq_chunks = [], [], []
    for k in range(N_CHUNKS):
        z_chunk = jax.lax.slice_in_dim(z_flat, k * CHUNK_ROWS,
                                       (k + 1) * CHUNK_ROWS, axis=0)
        idx2d, lpart = _vq_distances_argmin(z_chunk, codebook, csq)
        idx_chunks.append(idx2d)
        loss_parts.append(lpart)
        zq_chunks.append(_sc_gather(cb_padded, idx2d.reshape(1, CHUNK_ROWS)))

    loss = (sum(p[0, 0] for p in loss_parts) * (1.25 / (N_ROWS * EMB_DIM)))
    encoding_indices = jnp.concatenate(idx_chunks, axis=0).reshape(N_ROWS)
    z_q = jnp.concatenate([c[:, :EMB_DIM] for c in zq_chunks], axis=0)
    return z_q.reshape(z.shape), loss, encoding_indices


# speedup vs baseline: 1.2022x; 1.2022x over previous
import functools

import jax
import jax.numpy as jnp
from jax.experimental import pallas as pl
from jax.experimental.pallas import tpu as pltpu
from jax.experimental.pallas import tpu_sc as plsc

EMB_DIM = 64
NUM_CODES = 1024
N_ROWS = 16 * 1024
N_CHUNKS = 1
CHUNK_ROWS = N_ROWS // N_CHUNKS
ROW_TILE = 2048
GATHER_WINDOW = 128
GATHER_WIDTH = 128


def _vq_tc_body(n_tiles, z_ref, cb_ref, zsq_ref, csq_ref, idx_ref, loss_ref):
    z = z_ref[...]
    cb = cb_ref[...]
    dots = jax.lax.dot_general(z, cb, (((1,), (1,)), ((), ())),
                               preferred_element_type=jnp.float32)
    d2 = zsq_ref[...] + csq_ref[...] - 2.0 * dots
    dist = jnp.sqrt(jnp.clip(d2, 0.0, None))
    dmin = jnp.min(dist, axis=1, keepdims=True)
    ids = jax.lax.broadcasted_iota(jnp.int32, dist.shape, 1)
    idx = jnp.min(jnp.where(dist == dmin, ids, NUM_CODES), axis=1,
                  keepdims=True)
    idx_ref[...] = idx
    part = jnp.sum(dmin * dmin, keepdims=True)

    @pl.when(pl.program_id(0) == 0)
    def _init():
        loss_ref[...] = jnp.zeros_like(part)

    loss_ref[...] += part


def _vq_distances_argmin(z_chunk, codebook, csq):
    n_rows = z_chunk.shape[0]
    n_tiles = n_rows // ROW_TILE
    zsq = jnp.sum(z_chunk ** 2, axis=1, keepdims=True)
    return pl.pallas_call(
        functools.partial(_vq_tc_body, n_tiles),
        grid=(n_tiles,),
        in_specs=[
            pl.BlockSpec((ROW_TILE, EMB_DIM), lambda i: (i, 0)),
            pl.BlockSpec((NUM_CODES, EMB_DIM), lambda i: (0, 0)),
            pl.BlockSpec((ROW_TILE, 1), lambda i: (i, 0)),
            pl.BlockSpec((1, NUM_CODES), lambda i: (0, 0)),
        ],
        out_specs=[
            pl.BlockSpec((ROW_TILE, 1), lambda i: (i, 0)),
            pl.BlockSpec((1, 1), lambda i: (0, 0)),
        ],
        out_shape=[
            jax.ShapeDtypeStruct((n_rows, 1), jnp.int32),
            jax.ShapeDtypeStruct((1, 1), jnp.float32),
        ],
    )(z_chunk, codebook, zsq, csq)


def _sc_gather(codebook_padded, indices_2d):
    n_idx = indices_2d.shape[1]
    mesh = plsc.VectorSubcoreMesh(core_axis_name="core",
                                  subcore_axis_name="subcore")

    @pl.kernel(out_type=jax.ShapeDtypeStruct((n_idx, GATHER_WIDTH),
                                             jnp.float32),
               mesh=mesh)
    def gather_kernel(cb_hbm, i_hbm, o_hbm):
        def body(i_vmem, o_vmem):
            pltpu.sync_copy(cb_hbm.at[i_vmem.at[0]], o_vmem)

        pltpu.emit_pipeline(
            body,
            grid=(n_idx // GATHER_WINDOW,),
            in_specs=[pl.BlockSpec((1, GATHER_WINDOW),
                                   index_map=lambda i: (0, i))],
            out_specs=[pl.BlockSpec((GATHER_WINDOW, GATHER_WIDTH),
                                    index_map=lambda i: (i, 0))],
            core_axis_name=("core", "subcore"),
            dimension_semantics=(pltpu.PARALLEL,),
        )(i_hbm, o_hbm)

    return gather_kernel(codebook_padded, indices_2d)


def kernel(z, codebook):
    z_flat = z.reshape(-1, EMB_DIM)
    csq = jnp.sum(codebook ** 2, axis=1)[None, :]
    cb_padded = jnp.pad(codebook, ((0, 0), (0, GATHER_WIDTH - EMB_DIM)))

    idx_chunks, loss_parts, zq_chunks = [], [], []
    for k in range(N_CHUNKS):
        z_chunk = jax.lax.slice_in_dim(z_flat, k * CHUNK_ROWS,
                                       (k + 1) * CHUNK_ROWS, axis=0)
        idx2d, lpart = _vq_distances_argmin(z_chunk, codebook, csq)
        idx_chunks.append(idx2d)
        loss_parts.append(lpart)
        zq_chunks.append(_sc_gather(cb_padded, idx2d.reshape(1, CHUNK_ROWS)))

    loss = (sum(p[0, 0] for p in loss_parts) * (1.25 / (N_ROWS * EMB_DIM)))
    encoding_indices = jnp.concatenate(idx_chunks, axis=0).reshape(N_ROWS)
    z_q = jnp.concatenate([c[:, :EMB_DIM] for c in zq_chunks], axis=0)
    return z_q.reshape(z.shape), loss, encoding_indices
